# baseline (speedup 1.0000x reference)
import dataclasses
import functools

import jax
import jax.numpy as jnp
from jax import lax
from jax.experimental import pallas as pl
from jax.experimental.pallas import tpu as pltpu
from jax.experimental.pallas import tpu_sc as plsc

_NE = 64
_L = 16
_TILE = 128


def sc_partials(gate_logits, nworkers=32, tile=_TILE):
    nb, nt, ne = gate_logits.shape
    total = nb * nt
    tpw = total // nworkers
    rows_per_b = nt // tpw
    mesh = plsc.VectorSubcoreMesh(core_axis_name="c", subcore_axis_name="s")
    out_sd = jax.ShapeDtypeStruct((nworkers, _NE), jnp.float32)

    cp = pltpu.CompilerParams()
    if "needs_layout_passes" in pltpu.CompilerParams.__dataclass_fields__:
        cp = dataclasses.replace(cp, needs_layout_passes=False)

    @functools.partial(
        pl.kernel,
        mesh=mesh,
        compiler_params=cp,
        out_type=[out_sd, out_sd],
        scratch_types=[
            pltpu.VMEM((tile, _NE), jnp.float32),
            pltpu.VMEM((_NE,), jnp.float32),
            pltpu.VMEM((_NE,), jnp.float32),
            pltpu.SemaphoreType.DMA,
        ],
    )
    def k(x_hbm, pout, fout, tile_v, pacc, facc, sem):
        wid = lax.axis_index("s") * 2 + lax.axis_index("c")
        b = wid // rows_per_b
        r = (wid % rows_per_b) * tpw

        zeros = jnp.zeros((_L,), jnp.float32)
        for q in range(_NE // _L):
            pacc[pl.ds(q * _L, _L)] = zeros
            facc[pl.ds(q * _L, _L)] = zeros

        @pl.loop(0, tpw // tile)
        def _tiles(t):
            pltpu.async_copy(
                x_hbm.at[b].at[pl.ds(r + t * tile, tile)], tile_v, sem
            ).wait()

            @pl.loop(0, tile)
            def _tok(i):
                v = [tile_v[i, pl.ds(q * _L, _L)] for q in range(4)]
                sv = [plsc.sort_key_val(x, x, descending=True)[0] for x in v]
                ab = jnp.maximum(sv[0], lax.rev(sv[1], (0,)))
                cd = jnp.maximum(sv[2], lax.rev(sv[3], (0,)))
                ab = plsc.sort_key_val(ab, ab, descending=True)[0]
                cd = plsc.sort_key_val(cd, cd, descending=True)[0]
                w = jnp.maximum(ab, lax.rev(cd, (0,)))
                ws = plsc.sort_key_val(w, w, descending=True)[0]
                m = ws[0]
                t8 = ws[7]
                e = [jnp.exp(x - m) for x in v]
                s = jnp.sum(e[0] + e[1] + e[2] + e[3], axis=0)
                sv = jnp.full((_L,), s, jnp.float32)
                for q in range(4):
                    sl = pl.ds(q * _L, _L)
                    pacc[sl] = pacc[sl] + e[q] / sv
                    facc[sl] = facc[sl] + jnp.where(
                        v[q] >= t8, 1.0, 0.0
                    ).astype(jnp.float32)

        pltpu.sync_copy(pacc, pout.at[wid])
        pltpu.sync_copy(facc, fout.at[wid])

    return k(gate_logits)


def _sc_combine_body(pp_ref, ff_ref, loss_ref, *, total_tokens):
    p_i = jnp.sum(pp_ref[...], axis=0) / total_tokens
    f_i = jnp.sum(ff_ref[...], axis=0) / (total_tokens * 8)
    loss = 0.01 * _NE * jnp.sum(f_i * p_i)
    loss_ref[...] = jnp.full((1, 1), loss, jnp.float32)


def sc_loss(gate_logits):
    nb, nt, ne = gate_logits.shape
    total = nb * nt
    pp, ff = sc_partials(gate_logits)
    loss = pl.pallas_call(
        functools.partial(_sc_combine_body, total_tokens=float(total)),
        out_shape=jax.ShapeDtypeStruct((1, 1), jnp.float32),
    )(pp, ff)
    return loss[0, 0]


def kernel(gate_logits):
    return sc_loss(gate_logits)


# speedup vs baseline: 2.6338x; 2.6338x over previous
import dataclasses
import functools

import jax
import jax.numpy as jnp
from jax import lax
from jax.experimental import pallas as pl
from jax.experimental.pallas import tpu as pltpu
from jax.experimental.pallas import tpu_sc as plsc

_NUM_EXPERTS = 64
_TOP_K = 8
_ALPHA = 0.01
_LANES = 128
_SCL = 16
_SC_TOKENS_PER_ROW = 512

_NET = [(0, 1), (2, 3), (4, 5), (6, 7), (0, 2), (1, 3), (4, 6), (5, 7),
        (1, 2), (5, 6), (0, 4), (3, 7), (1, 5), (2, 6), (1, 4), (3, 6),
        (2, 4), (3, 5), (3, 4)]
_CLEAN = [(0, 4), (1, 5), (2, 6), (3, 7), (0, 2), (1, 3), (4, 6), (5, 7),
          (0, 1), (2, 3), (4, 5), (6, 7)]


def _ce(vs, net):
    for a, b in net:
        hi = jnp.maximum(vs[a], vs[b])
        lo = jnp.minimum(vs[a], vs[b])
        vs[a], vs[b] = hi, lo


def _tc_body(x_ref, pacc_ref, facc_ref, *, tc_tokens):
    @pl.when(pl.program_id(0) == 0)
    def _init():
        pacc_ref[...] = jnp.zeros_like(pacc_ref)
        facc_ref[...] = jnp.zeros_like(facc_ref)

    preg = None
    freg = None
    for j in range(tc_tokens // _LANES):
        xt = x_ref[0, j * _LANES : (j + 1) * _LANES, :].T

        s8 = [xt[8 * i : 8 * i + 8, :] for i in range(8)]
        _ce(s8, _NET)
        for d in (1, 2):
            rolled = [pltpu.roll(v, 8 - d, axis=0) for v in s8]
            s8 = [jnp.maximum(s8[i], rolled[7 - i]) for i in range(8)]
            _ce(s8, _CLEAN)
        rolled = [pltpu.roll(v, 4, axis=0) for v in s8]
        top8 = [jnp.maximum(s8[i], rolled[7 - i]) for i in range(8)]
        t8 = top8[0]
        gmax = top8[0]
        for i in range(1, 8):
            t8 = jnp.minimum(t8, top8[i])
            gmax = jnp.maximum(gmax, top8[i])
        t8 = t8[0:1, :]
        m1 = gmax[0:1, :]

        e = jnp.exp(xt - m1)
        s = jnp.sum(e, axis=0, keepdims=True)
        p = e / s
        mask = (xt >= t8).astype(jnp.float32)

        preg = p if preg is None else preg + p
        freg = mask if freg is None else freg + mask

    pacc_ref[...] += preg
    facc_ref[...] += freg


def _sc_partials(gate_logits):
    nb, nt, ne = gate_logits.shape
    nworkers = 32
    tpw = nb * _SC_TOKENS_PER_ROW // nworkers
    wpr = _SC_TOKENS_PER_ROW // tpw
    base = nt - _SC_TOKENS_PER_ROW
    mesh = plsc.VectorSubcoreMesh(core_axis_name="c", subcore_axis_name="s")
    out_sd = jax.ShapeDtypeStruct((nworkers, _NUM_EXPERTS), jnp.float32)

    cp = pltpu.CompilerParams()
    if "needs_layout_passes" in pltpu.CompilerParams.__dataclass_fields__:
        cp = dataclasses.replace(cp, needs_layout_passes=False)

    @functools.partial(
        pl.kernel,
        mesh=mesh,
        compiler_params=cp,
        out_type=[out_sd, out_sd],
        scratch_types=[
            pltpu.VMEM((tpw, _NUM_EXPERTS), jnp.float32),
            pltpu.VMEM((_NUM_EXPERTS,), jnp.float32),
            pltpu.VMEM((_NUM_EXPERTS,), jnp.float32),
            pltpu.SemaphoreType.DMA,
        ],
    )
    def k(x_hbm, pout, fout, tile_v, pacc, facc, sem):
        wid = lax.axis_index("s") * 2 + lax.axis_index("c")
        b = wid // wpr
        r = base + (wid % wpr) * tpw

        zeros = jnp.zeros((_SCL,), jnp.float32)
        for q in range(_NUM_EXPERTS // _SCL):
            pacc[pl.ds(q * _SCL, _SCL)] = zeros
            facc[pl.ds(q * _SCL, _SCL)] = zeros

        pltpu.async_copy(x_hbm.at[b].at[pl.ds(r, tpw)], tile_v, sem).wait()

        @pl.loop(0, tpw)
        def _tok(i):
            v = [tile_v[i, pl.ds(q * _SCL, _SCL)] for q in range(4)]
            sv = [plsc.sort_key_val(x, x, descending=True)[0] for x in v]
            ab = jnp.maximum(sv[0], lax.rev(sv[1], (0,)))
            cd = jnp.maximum(sv[2], lax.rev(sv[3], (0,)))
            ab = plsc.sort_key_val(ab, ab, descending=True)[0]
            cd = plsc.sort_key_val(cd, cd, descending=True)[0]
            w = jnp.maximum(ab, lax.rev(cd, (0,)))
            ws = plsc.sort_key_val(w, w, descending=True)[0]
            m = ws[0]
            t8 = ws[7]
            e = [jnp.exp(x - m) for x in v]
            s = jnp.sum(e[0] + e[1] + e[2] + e[3], axis=0)
            sv2 = jnp.full((_SCL,), s, jnp.float32)
            for q in range(4):
                sl = pl.ds(q * _SCL, _SCL)
                pacc[sl] = pacc[sl] + e[q] / sv2
                facc[sl] = facc[sl] + jnp.where(
                    v[q] >= t8, 1.0, 0.0
                ).astype(jnp.float32)

        pltpu.sync_copy(pacc, pout.at[wid])
        pltpu.sync_copy(facc, fout.at[wid])

    return k(gate_logits)


def _combine_body(pacc_ref, facc_ref, pp_ref, ff_ref, loss_ref, *, total_tokens):
    p_i = jnp.sum(pacc_ref[...], axis=1) + jnp.sum(pp_ref[...], axis=0)
    f_i = jnp.sum(facc_ref[...], axis=1) + jnp.sum(ff_ref[...], axis=0)
    p_i = p_i / total_tokens
    f_i = f_i / (total_tokens * _TOP_K)
    loss = _ALPHA * _NUM_EXPERTS * jnp.sum(f_i * p_i)
    loss_ref[...] = jnp.full((1, 1), loss, jnp.float32)


def kernel(gate_logits):
    nb, nt, ne = gate_logits.shape
    total = nb * nt
    tc_tokens = nt - _SC_TOKENS_PER_ROW

    pp, ff = _sc_partials(gate_logits)

    acc_shape = jax.ShapeDtypeStruct((_NUM_EXPERTS, _LANES), jnp.float32)
    pacc, facc = pl.pallas_call(
        functools.partial(_tc_body, tc_tokens=tc_tokens),
        grid=(nb,),
        in_specs=[pl.BlockSpec((1, nt, ne), lambda i: (i, 0, 0))],
        out_specs=[
            pl.BlockSpec((_NUM_EXPERTS, _LANES), lambda i: (0, 0)),
            pl.BlockSpec((_NUM_EXPERTS, _LANES), lambda i: (0, 0)),
        ],
        out_shape=[acc_shape, acc_shape],
    )(gate_logits)

    loss = pl.pallas_call(
        functools.partial(_combine_body, total_tokens=float(total)),
        out_shape=jax.ShapeDtypeStruct((1, 1), jnp.float32),
    )(pacc, facc, pp, ff)
    return loss[0, 0]


# speedup vs baseline: 4.2116x; 1.5991x over previous
import functools

import jax
import jax.numpy as jnp
from jax.experimental import pallas as pl
from jax.experimental.pallas import tpu as pltpu

_NUM_EXPERTS = 64
_TOP_K = 8
_ALPHA = 0.01
_LANES = 128

_NET = [(0, 1), (2, 3), (4, 5), (6, 7), (0, 2), (1, 3), (4, 6), (5, 7),
        (1, 2), (5, 6), (0, 4), (3, 7), (1, 5), (2, 6), (1, 4), (3, 6),
        (2, 4), (3, 5), (3, 4)]
_CLEAN = [(0, 4), (1, 5), (2, 6), (3, 7), (0, 2), (1, 3), (4, 6), (5, 7),
          (0, 1), (2, 3), (4, 5), (6, 7)]


def _ce(vs, net):
    for a, b in net:
        hi = jnp.maximum(vs[a], vs[b])
        lo = jnp.minimum(vs[a], vs[b])
        vs[a], vs[b] = hi, lo


def _body(x_ref, loss_ref, pacc_ref, facc_ref, *, nsteps, total_tokens):
    step = pl.program_id(0)

    @pl.when(step == 0)
    def _init():
        pacc_ref[...] = jnp.zeros_like(pacc_ref)
        facc_ref[...] = jnp.zeros_like(facc_ref)

    rows, block = x_ref.shape[0], x_ref.shape[1]
    preg = None
    freg = None
    for rj in range(rows * (block // _LANES)):
        r, j = divmod(rj, block // _LANES)
        xt = x_ref[r, j * _LANES : (j + 1) * _LANES, :].T

        s8 = [xt[8 * i : 8 * i + 8, :] for i in range(8)]
        _ce(s8, _NET)
        for d in (1, 2):
            rolled = [pltpu.roll(v, 8 - d, axis=0) for v in s8]
            s8 = [jnp.maximum(s8[i], rolled[7 - i]) for i in range(8)]
            _ce(s8, _CLEAN)
        rolled = [pltpu.roll(v, 4, axis=0) for v in s8]
        top8 = [jnp.maximum(s8[i], rolled[7 - i]) for i in range(8)]
        t8 = top8[0]
        gmax = top8[0]
        for i in range(1, 8):
            t8 = jnp.minimum(t8, top8[i])
            gmax = jnp.maximum(gmax, top8[i])
        t8 = t8[0:1, :]
        m1 = gmax[0:1, :]

        e = jnp.exp(xt - m1)
        s = jnp.sum(e, axis=0, keepdims=True)
        p = e / s
        mask = (xt >= t8).astype(jnp.float32)

        preg = p if preg is None else preg + p
        freg = mask if freg is None else freg + mask

    pacc_ref[...] += preg
    facc_ref[...] += freg

    @pl.when(step == nsteps - 1)
    def _finish():
        p_i = jnp.sum(pacc_ref[...], axis=1) / total_tokens
        f_i = jnp.sum(facc_ref[...], axis=1) / (total_tokens * _TOP_K)
        loss = _ALPHA * _NUM_EXPERTS * jnp.sum(f_i * p_i)
        loss_ref[...] = jnp.full((1, 1), loss, jnp.float32)


def kernel(gate_logits):
    nb, nt, ne = gate_logits.shape
    total = nb * nt
    loss = pl.pallas_call(
        functools.partial(_body, nsteps=nb // 2, total_tokens=float(total)),
        grid=(nb // 2,),
        in_specs=[pl.BlockSpec((2, nt, ne), lambda i: (i, 0, 0))],
        out_specs=pl.BlockSpec((1, 1), lambda i: (0, 0)),
        out_shape=jax.ShapeDtypeStruct((1, 1), jnp.float32),
        scratch_shapes=[
            pltpu.VMEM((_NUM_EXPERTS, _LANES), jnp.float32),
            pltpu.VMEM((_NUM_EXPERTS, _LANES), jnp.float32),
        ],
    )(gate_logits)
    return loss[0, 0]


# speedup vs baseline: 4.3699x; 1.0376x over previous
import functools

import jax
import jax.numpy as jnp
from jax.experimental import pallas as pl
from jax.experimental.pallas import tpu as pltpu

_NUM_EXPERTS = 64
_TOP_K = 8
_ALPHA = 0.01
_LANES = 128

_NET = [(0, 1), (2, 3), (4, 5), (6, 7), (0, 2), (1, 3), (4, 6), (5, 7),
        (1, 2), (5, 6), (0, 4), (3, 7), (1, 5), (2, 6), (1, 4), (3, 6),
        (2, 4), (3, 5), (3, 4)]
_CLEAN = [(0, 4), (1, 5), (2, 6), (3, 7), (0, 2), (1, 3), (4, 6), (5, 7),
          (0, 1), (2, 3), (4, 5), (6, 7)]


def _ce(vs, net):
    for a, b in net:
        hi = jnp.maximum(vs[a], vs[b])
        lo = jnp.minimum(vs[a], vs[b])
        vs[a], vs[b] = hi, lo


def _body(x_ref, loss_ref, pacc_ref, facc_ref, *, nsteps, total_tokens):
    step = pl.program_id(0)

    @pl.when(step == 0)
    def _init():
        pacc_ref[...] = jnp.zeros_like(pacc_ref)
        facc_ref[...] = jnp.zeros_like(facc_ref)

    block = x_ref.shape[1]
    preg = None
    freg = None
    for j in range(block // _LANES):
        xt = x_ref[0, j * _LANES : (j + 1) * _LANES, :].T

        s8 = [xt[8 * i : 8 * i + 8, :] for i in range(8)]
        _ce(s8, _NET)
        for d in (1, 2):
            rolled = [pltpu.roll(v, 8 - d, axis=0) for v in s8]
            s8 = [jnp.maximum(s8[i], rolled[7 - i]) for i in range(8)]
            _ce(s8, _CLEAN)
        rolled = [pltpu.roll(v, 4, axis=0) for v in s8]
        top8 = [jnp.maximum(s8[i], rolled[7 - i]) for i in range(8)]
        t8 = top8[0]
        gmax = top8[0]
        for i in range(1, 8):
            t8 = jnp.minimum(t8, top8[i])
            gmax = jnp.maximum(gmax, top8[i])
        t8 = t8[0:1, :]
        m1 = gmax[0:1, :]

        e = jnp.exp(xt - m1)
        s = jnp.sum(e, axis=0, keepdims=True)
        p = e / s
        mask = (xt >= t8).astype(jnp.float32)

        preg = p if preg is None else preg + p
        freg = mask if freg is None else freg + mask

    pacc_ref[...] += preg
    facc_ref[...] += freg

    @pl.when(step == nsteps - 1)
    def _finish():
        p_i = jnp.sum(pacc_ref[...], axis=1) / total_tokens
        f_i = jnp.sum(facc_ref[...], axis=1) / (total_tokens * _TOP_K)
        loss = _ALPHA * _NUM_EXPERTS * jnp.sum(f_i * p_i)
        loss_ref[...] = jnp.full((1, 1), loss, jnp.float32)


def kernel(gate_logits):
    nb, nt, ne = gate_logits.shape
    total = nb * nt
    loss = pl.pallas_call(
        functools.partial(_body, nsteps=nb, total_tokens=float(total)),
        grid=(nb,),
        in_specs=[pl.BlockSpec((1, nt, ne), lambda i: (i, 0, 0))],
        out_specs=pl.BlockSpec((1, 1), lambda i: (0, 0)),
        out_shape=jax.ShapeDtypeStruct((1, 1), jnp.float32),
        scratch_shapes=[
            pltpu.VMEM((_NUM_EXPERTS, _LANES), jnp.float32),
            pltpu.VMEM((_NUM_EXPERTS, _LANES), jnp.float32),
        ],
    )(gate_logits)
    return loss[0, 0]
